# Initial kernel scaffold; baseline (speedup 1.0000x reference)
import jax
import jax.numpy as jnp
from jax.experimental import pallas as pl


def kernel(emb_diag, emb_proc, emb_med, gamma_diag, beta_diag, gamma_proc, beta_proc, gamma_med, beta_med, w_diag, w_proc, w_med, node_idx_diag, edge_idx_diag, node_idx_proc, edge_idx_proc, node_idx_med, edge_idx_med):
    raise NotImplementedError("write your pallas kernel here")


# speedup vs baseline: 3.1520x; 3.1520x over previous
import functools

import jax
import jax.numpy as jnp
from jax import lax
from jax.experimental import pallas as pl
from jax.experimental.pallas import tpu as pltpu
from jax.experimental.pallas import tpu_sc as plsc

N_EDGES = 50000
D = 64
NC = 2
NS = 16
NW = NC * NS
SUB = 400
SUBS_PER_TILE = 4
NSUB = NW * SUBS_PER_TILE
E_PAD = NSUB * SUB
CHUNK = 128
STEP = 112


def _coef_body(ed, ep, em, gd, bd, gp, bp, gm, bm, out_ref):
    tabs = ((ed, 10000, gd, bd), (ep, 10000, gp, bp), (em, 5000, gm, bm))
    for t, (ref, n, g, b) in enumerate(tabs):
        x = ref[0:n, :]
        mu = jnp.mean(x, axis=0, keepdims=True)
        var = jnp.mean((x - mu) ** 2, axis=0, keepdims=True)
        a = g[...] * lax.rsqrt(var + 1e-5)
        bb = b[...] - mu * a
        out_ref[2 * t:2 * t + 1, :] = a
        out_ref[2 * t + 1:2 * t + 2, :] = bb


def _coef(ed, ep, em, gd, bd, gp, bp, gm, bm):
    return pl.pallas_call(
        _coef_body,
        out_shape=jax.ShapeDtypeStruct((6, D), jnp.float32),
    )(ed, ep, em, gd.reshape(1, D), bd.reshape(1, D),
      gp.reshape(1, D), bp.reshape(1, D), gm.reshape(1, D), bm.reshape(1, D))


def _sc_body(bounds_hbm, coef_hbm, emb_d, emb_p, emb_m,
             nidx_d, eidx_d, w_d, nidx_p, eidx_p, w_p, nidx_m, eidx_m, w_m,
             out_dp, out_m,
             bounds_v, coef_v, nbuf, ebuf, wbuf, rows, acc, den, outb, sem):
    wid = lax.axis_index("s") * NC + lax.axis_index("c")
    pltpu.sync_copy(bounds_hbm, bounds_v)
    pltpu.sync_copy(coef_hbm, coef_v)

    fzero = jnp.zeros((16,), jnp.float32)
    iota = lax.broadcasted_iota(jnp.int32, (16,), 0)

    tabs = ((emb_d, nidx_d, eidx_d, w_d, 800000, 0),
            (emb_p, nidx_p, eidx_p, w_p, 800000, 1),
            (emb_m, nidx_m, eidx_m, w_m, 400000, 2))

    def qbody(q, _):
        s = wid * SUBS_PER_TILE + q
        row0 = s * SUB
        for (emb, nidx, eidx, w, nnz, t) in tabs:
            def zbody(r, _):
                den[r, :] = fzero
                for j in range(4):
                    acc[r, pl.ds(j * 16, 16)] = fzero
                return 0
            lax.fori_loop(0, SUB, zbody, 0)

            bv = bounds_v[t, s, :]
            lo = bv[0]
            hi = bv[1]
            nwin = (hi - lo + (STEP - 1)) // STEP

            def wbody(k, _, emb=emb, nidx=nidx, eidx=eidx, w=w, nnz=nnz,
                      lo=lo, hi=hi, row0=row0):
                start = lo + k * STEP
                base = jnp.minimum((start // 8) * 8, nnz - CHUNK)
                end = jnp.minimum(start + STEP, hi)
                pltpu.sync_copy(nidx.at[pl.ds(base, CHUNK)], nbuf)
                pltpu.sync_copy(eidx.at[pl.ds(base, CHUNK)], ebuf)
                pltpu.sync_copy(w.at[pl.ds(base, CHUNK)], wbuf)
                pltpu.async_copy(emb.at[nbuf], rows, sem).wait()
                g0 = (start - base) // 16
                g1 = (end - base + 15) // 16

                def gbody(g, _):
                    off = pl.multiple_of(g * 16, 16)
                    iv = base + off + iota
                    mask = (iv >= start) & (iv < end)
                    ev = ebuf[pl.ds(off, 16)] - row0
                    ec = jnp.minimum(jnp.maximum(ev, 0), SUB - 1)
                    wv = jnp.where(mask, wbuf[pl.ds(off, 16)], 0.0)
                    for u in range(16):
                        e_u = ec[u]
                        w_u = wv[u]
                        ri = off + u
                        plsc.addupdate(den.at[e_u, :], jnp.full((16,), w_u))
                        for j in range(4):
                            r16 = rows[ri, pl.ds(j * 16, 16)]
                            plsc.addupdate(acc.at[e_u, pl.ds(j * 16, 16)],
                                           r16 * w_u)
                    return 0
                lax.fori_loop(g0, g1, gbody, 0)
                return 0
            lax.fori_loop(0, nwin, wbody, 0)

            def dbody(r, _, t=t):
                dv = den[r, :]
                dsafe = dv + 1e-12
                f = dv / dsafe
                for j in range(4):
                    sl = pl.ds(j * 16, 16)
                    c = coef_v[2 * t, sl] * (acc[r, sl] / dsafe) \
                        + coef_v[2 * t + 1, sl] * f
                    if t == 1:
                        outb[r, sl] = outb[r, sl] + c
                    else:
                        outb[r, sl] = c
                return 0
            lax.fori_loop(0, SUB, dbody, 0)

            if t == 1:
                pltpu.sync_copy(outb, out_dp.at[pl.ds(row0, SUB)])
            if t == 2:
                pltpu.sync_copy(outb, out_m.at[pl.ds(row0, SUB)])
        return 0
    lax.fori_loop(0, SUBS_PER_TILE, qbody, 0)


_sc_agg = functools.partial(
    pl.kernel,
    mesh=plsc.VectorSubcoreMesh(core_axis_name="c", subcore_axis_name="s"),
    compiler_params=pltpu.CompilerParams(use_tc_tiling_on_sc=False),
    out_type=[jax.ShapeDtypeStruct((E_PAD, D), jnp.float32),
              jax.ShapeDtypeStruct((E_PAD, D), jnp.float32)],
    scratch_types=[
        pltpu.VMEM((3, NSUB, 16), jnp.int32),
        pltpu.VMEM((6, D), jnp.float32),
        pltpu.VMEM((CHUNK,), jnp.int32),
        pltpu.VMEM((CHUNK,), jnp.int32),
        pltpu.VMEM((CHUNK,), jnp.float32),
        pltpu.VMEM((CHUNK, D), jnp.float32),
        pltpu.VMEM((SUB, D), jnp.float32),
        pltpu.VMEM((SUB, 16), jnp.float32),
        pltpu.VMEM((SUB, D), jnp.float32),
        pltpu.SemaphoreType.DMA,
    ],
)(_sc_body)


def kernel(emb_diag, emb_proc, emb_med, gamma_diag, beta_diag, gamma_proc,
           beta_proc, gamma_med, beta_med, w_diag, w_proc, w_med,
           node_idx_diag, edge_idx_diag, node_idx_proc, edge_idx_proc,
           node_idx_med, edge_idx_med):
    coef = _coef(emb_diag, emb_proc, emb_med, gamma_diag, beta_diag,
                 gamma_proc, beta_proc, gamma_med, beta_med)
    ss = SUB * jnp.arange(NSUB + 1, dtype=jnp.int32)
    starts = jnp.stack([
        jnp.searchsorted(edge_idx_diag, ss).astype(jnp.int32),
        jnp.searchsorted(edge_idx_proc, ss).astype(jnp.int32),
        jnp.searchsorted(edge_idx_med, ss).astype(jnp.int32),
    ])
    bounds = jnp.stack([starts[:, :-1], starts[:, 1:]], axis=-1)
    bounds = jnp.pad(bounds, ((0, 0), (0, 0), (0, 14)))
    out_dp, out_m = _sc_agg(
        bounds, coef, emb_diag, emb_proc, emb_med,
        node_idx_diag, edge_idx_diag, w_diag,
        node_idx_proc, edge_idx_proc, w_proc,
        node_idx_med, edge_idx_med, w_med)
    return jnp.concatenate([out_dp[:N_EDGES], out_m[:N_EDGES]], axis=0)


# speedup vs baseline: 9.1461x; 2.9017x over previous
import functools

import jax
import jax.numpy as jnp
from jax import lax
from jax.experimental import pallas as pl
from jax.experimental.pallas import tpu as pltpu
from jax.experimental.pallas import tpu_sc as plsc

N_EDGES = 50000
D = 64
NC = 2
NS = 16
NW = NC * NS
SUB = 400
SUBS_PER_TILE = 4
NSUB = NW * SUBS_PER_TILE
NSUB_LIVE = N_EDGES // SUB
IBLK = 1024
WIN = 128
NOUT = 2 * N_EDGES


def _coef_body(ed, ep, em, gd, bd, gp, bp, gm, bm, out_ref):
    tabs = ((ed, 10000, gd, bd), (ep, 10000, gp, bp), (em, 5000, gm, bm))
    for t, (ref, n, g, b) in enumerate(tabs):
        x = ref[0:n, :]
        mu = jnp.mean(x, axis=0, keepdims=True)
        var = jnp.mean((x - mu) ** 2, axis=0, keepdims=True)
        a = g[...] * lax.rsqrt(var + 1e-5)
        bb = b[...] - mu * a
        out_ref[2 * t:2 * t + 1, :] = a
        out_ref[2 * t + 1:2 * t + 2, :] = bb


def _coef(ed, ep, em, gd, bd, gp, bp, gm, bm):
    return pl.pallas_call(
        _coef_body,
        out_shape=jax.ShapeDtypeStruct((6, D), jnp.float32),
    )(ed, ep, em, gd.reshape(1, D), bd.reshape(1, D),
      gp.reshape(1, D), bp.reshape(1, D), gm.reshape(1, D), bm.reshape(1, D))


def _sc_body(bounds_hbm, coef_hbm, emb_d, emb_p, emb_m,
             nidx_d, eidx_d, w_d, nidx_p, eidx_p, w_p, nidx_m, eidx_m, w_m,
             out,
             bounds_v, coef_v, nbuf, ebuf, wbuf, rows0, rows1, rows2, rows3,
             acc, den, outb, semi, sem0, sem1, sem2, sem3):
    wid = lax.axis_index("s") * NC + lax.axis_index("c")
    pltpu.sync_copy(bounds_hbm, bounds_v)
    pltpu.sync_copy(coef_hbm, coef_v)

    fzero = jnp.zeros((16,), jnp.float32)
    iota = lax.broadcasted_iota(jnp.int32, (16,), 0)

    tabs = ((emb_d, nidx_d, eidx_d, w_d, 800000, 0),
            (emb_p, nidx_p, eidx_p, w_p, 800000, 1),
            (emb_m, nidx_m, eidx_m, w_m, 400000, 2))

    def z0(r, _):
        den[r, :] = fzero
        for j in range(4):
            acc[r, pl.ds(j * 16, 16)] = fzero
        return 0
    lax.fori_loop(0, SUB, z0, 0)

    def qbody(q, _):
        s = wid * SUBS_PER_TILE + q
        row0 = s * SUB

        @pl.when(s < NSUB_LIVE)
        def _():
            for (emb, nidx, eidx, w, nnz, t) in tabs:
                bv = bounds_v[t, s, :]
                lo = bv[0]
                hi = bv[1]
                base0 = (lo // 8) * 8
                nblk = (hi - base0 + IBLK - 1) // IBLK

                def bbody(b, _, emb=emb, nidx=nidx, eidx=eidx, w=w,
                          nnz=nnz, lo=lo, hi=hi, row0=row0, base0=base0):
                    bnom = base0 + b * IBLK
                    bst = jnp.minimum(bnom, nnz - IBLK)
                    blk_lo = jnp.maximum(lo, bnom) - bst
                    blk_hi = jnp.minimum(hi, bnom + IBLK) - bst
                    cn = pltpu.async_copy(nidx.at[pl.ds(bst, IBLK)], nbuf,
                                          semi)
                    ce = pltpu.async_copy(eidx.at[pl.ds(bst, IBLK)], ebuf,
                                          semi)
                    cw = pltpu.async_copy(w.at[pl.ds(bst, IBLK)], wbuf, semi)
                    cn.wait()
                    ce.wait()
                    cw.wait()
                    g0 = blk_lo // 16
                    g1 = (blk_hi + 15) // 16
                    j0 = g0 // 8
                    j1 = (g1 + 7) // 8
                    njw = j1 - j0

                    def issue(j, buf, sem):
                        return pltpu.async_copy(
                            emb.at[nbuf.at[pl.ds(pl.multiple_of(j * WIN, WIN),
                                                 WIN)]],
                            buf, sem)

                    def process(j, rows, g0=g0, g1=g1, blk_lo=blk_lo,
                                blk_hi=blk_hi, row0=row0):
                        ga = jnp.maximum(g0, j * 8)
                        gb = jnp.minimum(g1, j * 8 + 8)

                        def gbody(g, _):
                            off = pl.multiple_of(g * 16, 16)
                            lid = off + iota
                            mask = (lid >= blk_lo) & (lid < blk_hi)
                            ev = ebuf[pl.ds(off, 16)] - row0
                            ec = jnp.minimum(jnp.maximum(ev, 0), SUB - 1)
                            wv = jnp.where(mask, wbuf[pl.ds(off, 16)], 0.0)
                            roff = off - j * WIN

                            def loads(u):
                                wu = jnp.full((16,), wv[u])
                                r = [rows[roff + u, pl.ds(jj * 16, 16)]
                                     for jj in range(4)]
                                return wu, r

                            def stores(u, wu, r):
                                e_u = ec[u]
                                plsc.addupdate(den.at[e_u, :], wu)
                                for jj in range(4):
                                    plsc.addupdate(
                                        acc.at[e_u, pl.ds(jj * 16, 16)],
                                        r[jj] * wu)

                            wu, r = loads(0)
                            for u in range(1, 16):
                                wu2, r2 = loads(u)
                                stores(u - 1, wu, r)
                                wu, r = wu2, r2
                            stores(15, wu, r)
                            return 0
                        lax.fori_loop(ga, gb, gbody, 0)

                    def drain(rows, sem, emb=emb):
                        pltpu.make_async_copy(emb.at[pl.ds(0, WIN)], rows,
                                              sem).wait()

                    slots = ((rows0, sem0), (rows1, sem1),
                             (rows2, sem2), (rows3, sem3))
                    nd = len(slots)

                    @pl.when(njw > 0)
                    def _():
                        for d, (buf, sem) in enumerate(slots):
                            @pl.when(njw > d)
                            def _(d=d, buf=buf, sem=sem):
                                issue(j0 + d, buf, sem)

                        def pbody(p, _):
                            for d, (buf, sem) in enumerate(slots):
                                jA = j0 + nd * p + d

                                @pl.when(jA < j1)
                                def _(jA=jA, buf=buf, sem=sem):
                                    drain(buf, sem)
                                    process(jA, buf)

                                    @pl.when(jA + nd < j1)
                                    def _(jA=jA, buf=buf, sem=sem):
                                        issue(jA + nd, buf, sem)
                            return 0
                        lax.fori_loop(0, (njw + nd - 1) // nd, pbody, 0)
                    return 0
                lax.fori_loop(0, nblk, bbody, 0)

                def dbody(r, _, t=t):
                    dv = den[r, :]
                    rinv = 1.0 / (dv + 1e-12)
                    f = dv * rinv
                    den[r, :] = fzero
                    for j in range(4):
                        sl = pl.ds(j * 16, 16)
                        c = coef_v[2 * t, sl] * (acc[r, sl] * rinv) \
                            + coef_v[2 * t + 1, sl] * f
                        acc[r, sl] = fzero
                        if t == 1:
                            outb[r, sl] = outb[r, sl] + c
                        else:
                            outb[r, sl] = c
                    return 0
                lax.fori_loop(0, SUB, dbody, 0)

                if t == 1:
                    pltpu.sync_copy(outb, out.at[pl.ds(row0, SUB)])
                if t == 2:
                    pltpu.sync_copy(outb, out.at[pl.ds(N_EDGES + row0, SUB)])
        return 0
    lax.fori_loop(0, SUBS_PER_TILE, qbody, 0)


_sc_agg = functools.partial(
    pl.kernel,
    mesh=plsc.VectorSubcoreMesh(core_axis_name="c", subcore_axis_name="s"),
    compiler_params=pltpu.CompilerParams(use_tc_tiling_on_sc=False),
    out_type=jax.ShapeDtypeStruct((NOUT, D), jnp.float32),
    scratch_types=[
        pltpu.VMEM((3, NSUB, 16), jnp.int32),
        pltpu.VMEM((6, D), jnp.float32),
        pltpu.VMEM((IBLK,), jnp.int32),
        pltpu.VMEM((IBLK,), jnp.int32),
        pltpu.VMEM((IBLK,), jnp.float32),
        pltpu.VMEM((WIN, D), jnp.float32),
        pltpu.VMEM((WIN, D), jnp.float32),
        pltpu.VMEM((WIN, D), jnp.float32),
        pltpu.VMEM((WIN, D), jnp.float32),
        pltpu.VMEM((SUB, D), jnp.float32),
        pltpu.VMEM((SUB, 16), jnp.float32),
        pltpu.VMEM((SUB, D), jnp.float32),
        pltpu.SemaphoreType.DMA,
        pltpu.SemaphoreType.DMA,
        pltpu.SemaphoreType.DMA,
        pltpu.SemaphoreType.DMA,
        pltpu.SemaphoreType.DMA,
    ],
)(_sc_body)


def kernel(emb_diag, emb_proc, emb_med, gamma_diag, beta_diag, gamma_proc,
           beta_proc, gamma_med, beta_med, w_diag, w_proc, w_med,
           node_idx_diag, edge_idx_diag, node_idx_proc, edge_idx_proc,
           node_idx_med, edge_idx_med):
    coef = _coef(emb_diag, emb_proc, emb_med, gamma_diag, beta_diag,
                 gamma_proc, beta_proc, gamma_med, beta_med)
    ss = SUB * jnp.arange(NSUB + 1, dtype=jnp.int32)
    starts = jnp.stack([
        jnp.searchsorted(edge_idx_diag, ss).astype(jnp.int32),
        jnp.searchsorted(edge_idx_proc, ss).astype(jnp.int32),
        jnp.searchsorted(edge_idx_med, ss).astype(jnp.int32),
    ])
    bounds = jnp.stack([starts[:, :-1], starts[:, 1:]], axis=-1)
    bounds = jnp.pad(bounds, ((0, 0), (0, 0), (0, 14)))
    return _sc_agg(
        bounds, coef, emb_diag, emb_proc, emb_med,
        node_idx_diag, edge_idx_diag, w_diag,
        node_idx_proc, edge_idx_proc, w_proc,
        node_idx_med, edge_idx_med, w_med)
